# Initial kernel scaffold; baseline (speedup 1.0000x reference)
import jax
import jax.numpy as jnp
from jax.experimental import pallas as pl


def kernel(x, edge_index, edge_attr, W_src, W_dst, att, W_edge, bias, W_mid):
    raise NotImplementedError("write your pallas kernel here")


# speedup vs baseline: 5.5602x; 5.5602x over previous
import functools
import math

import jax
import jax.numpy as jnp
from jax import lax
from jax.experimental import pallas as pl
from jax.experimental.pallas import tpu as pltpu
from jax.experimental.pallas import tpu_sc as plsc

N = 10000
E = 320000
D = 128
ED = 16
NCONV = 5
NMID = 3
ALPHA = 0.2
THETA = 0.2

NC = 2
NS = 16
NW = NC * NS

NPAD = 10240
EPT = E // NS
NSL = NPAD // NS

BE = 2560
CH = 2000
NT = E // CH
CPT = 4

_MESH = plsc.VectorSubcoreMesh(core_axis_name="c", subcore_axis_name="s",
                               num_cores=NC, num_subcores=NS)
_SC_PARAMS = pltpu.CompilerParams(needs_layout_passes=False)


def _edge_pre_body(ea_ref, att_ref, we_ref, o_ref):
    rows = [jnp.dot(att_ref[i:i + 1, :], we_ref[i],
                    preferred_element_type=jnp.float32)
            for i in range(NCONV)]
    u = jnp.concatenate(rows, axis=0)
    o_ref[...] = lax.dot_general(u, ea_ref[...], (((1,), (1,)), ((), ())),
                                 preferred_element_type=jnp.float32)


def _tc_edge_pre(edge_attr, att_s, W_edge):
    return pl.pallas_call(
        _edge_pre_body,
        grid=(E // BE,),
        in_specs=[
            pl.BlockSpec((BE, ED), lambda e: (e, 0)),
            pl.BlockSpec((NCONV, D), lambda e: (0, 0)),
            pl.BlockSpec((NCONV, D, ED), lambda e: (0, 0, 0)),
        ],
        out_specs=pl.BlockSpec((NCONV, BE), lambda e: (0, e)),
        out_shape=jax.ShapeDtypeStruct((NCONV, E), jnp.float32),
    )(edge_attr, att_s, W_edge)


def _transpose_body(x_ref, eye_ref, o_ref):
    o_ref[...] = lax.dot_general(eye_ref[...], x_ref[...],
                                 (((1,), (1,)), ((), ())),
                                 preferred_element_type=jnp.float32)


def _tc_transpose(x, eye):
    return pl.pallas_call(
        _transpose_body,
        out_shape=jax.ShapeDtypeStruct((D, N), jnp.float32),
    )(x, eye)


def _matmul_body(h_ref, ws_ref, wd_ref, as_ref, ad_ref,
                 xs_ref, ss_ref, sd_ref):
    hb = h_ref[...]
    xs = jnp.dot(ws_ref[...], hb, preferred_element_type=jnp.float32)
    xs_ref[...] = xs
    ss_ref[...] = jnp.dot(as_ref[...], xs, preferred_element_type=jnp.float32)
    wd = jnp.dot(ad_ref[...], wd_ref[...], preferred_element_type=jnp.float32)
    sd_ref[...] = jnp.dot(wd, hb, preferred_element_type=jnp.float32)


def _tc_matmul(H, Wsrc, Wdst, att_s_i, att_d_i):
    return pl.pallas_call(
        _matmul_body,
        out_shape=[
            jax.ShapeDtypeStruct((D, N), jnp.float32),
            jax.ShapeDtypeStruct((1, N), jnp.float32),
            jax.ShapeDtypeStruct((1, N), jnp.float32),
        ],
    )(H, Wsrc, Wdst, att_s_i, att_d_i)


def _bias_body(o_ref, b_ref, y_ref):
    y_ref[...] = o_ref[...] + b_ref[...]


def _tc_bias(OT, bias_col):
    return pl.pallas_call(
        _bias_body,
        out_shape=jax.ShapeDtypeStruct((D, N), jnp.float32),
    )(OT, bias_col)


def _resid_body(beta, o_ref, x0_ref, w_ref, b_ref, y_ref):
    c = o_ref[...] + b_ref[...]
    h2 = c * (1.0 - ALPHA) + ALPHA * x0_ref[...]
    hm = lax.dot_general(w_ref[...], h2, (((0,), (0,)), ((), ())),
                         preferred_element_type=jnp.float32)
    h = (1.0 - beta) * h2 + beta * hm
    y_ref[...] = jnp.where(h > 0, h, 0.01 * h)


def _tc_resid(OT, X0T, Wmid, bias_col, beta):
    return pl.pallas_call(
        functools.partial(_resid_body, beta),
        out_shape=jax.ShapeDtypeStruct((D, N), jnp.float32),
    )(OT, X0T, Wmid, bias_col)


def _final_body(o_ref, b_ref, eye_ref, y_ref):
    y = lax.dot_general(o_ref[...], eye_ref[...], (((0,), (0,)), ((), ())),
                        preferred_element_type=jnp.float32)
    y_ref[...] = y + b_ref[...]


def _tc_final(OT, bias_row, eye):
    return pl.pallas_call(
        _final_body,
        out_shape=jax.ShapeDtypeStruct((N, D), jnp.float32),
    )(OT, bias_row, eye)


def _sc_scalar_body(src_h, dst_h, ssrc_h, sdst_h, ep_h, an_h,
                    ssrc_v, sdst_v, denom_v, src_v, dst_v, aexp_v,
                    mvec_v, allmax_v, slice_v, part16_v,
                    max_sh, part_sh, denom_sh):
    c = lax.axis_index("c")
    s = lax.axis_index("s")
    eb = s * EPT

    pltpu.sync_copy(src_h.at[pl.ds(eb, EPT)], src_v)
    pltpu.sync_copy(dst_h.at[pl.ds(eb, EPT)], dst_v)
    pltpu.sync_copy(ep_h.at[pl.ds(eb, EPT)], aexp_v)
    pltpu.sync_copy(ssrc_h, ssrc_v)
    pltpu.sync_copy(sdst_h, sdst_v)

    def zero(k, _):
        denom_v[pl.ds(k * 16, 16)] = jnp.zeros((16,), jnp.float32)
        return 0
    lax.fori_loop(0, NPAD // 16, zero, 0)

    def logits(sl):
        a = (plsc.load_gather(ssrc_v, [src_v[sl]])
             + plsc.load_gather(sdst_v, [dst_v[sl]])
             + aexp_v[sl])
        return jnp.where(a > 0, a, 0.2 * a)

    def p1(k, m):
        return jnp.maximum(m, logits(pl.ds(k * 16, 16)))
    m = lax.fori_loop(0, EPT // 16, p1,
                      jnp.full((16,), -1e30, jnp.float32))
    mvec_v[...] = m
    pltpu.sync_copy(mvec_v, max_sh.at[s])
    plsc.subcore_barrier()
    pltpu.sync_copy(max_sh, allmax_v)

    def mx(r, m):
        return jnp.maximum(m, allmax_v[r, :])
    m = lax.fori_loop(0, NS, mx, jnp.full((16,), -1e30, jnp.float32))
    M = jnp.max(m)

    def p2(k, _):
        sl = pl.ds(k * 16, 16)
        e = jnp.exp(logits(sl) - M)
        aexp_v[sl] = e
        plsc.addupdate_scatter(denom_v, [dst_v[sl]], e)
        return 0
    lax.fori_loop(0, EPT // 16, p2, 0)

    pltpu.sync_copy(denom_v, part_sh.at[s])
    plsc.subcore_barrier()
    pltpu.sync_copy(part_sh.at[:, pl.ds(s * NSL, NSL)], part16_v)

    def csum(k, _):
        sl = pl.ds(k * 16, 16)
        acc = part16_v[0, sl]
        for r in range(1, NS):
            acc = acc + part16_v[r, sl]
        slice_v[sl] = acc
        return 0
    lax.fori_loop(0, NSL // 16, csum, 0)
    pltpu.sync_copy(slice_v, denom_sh.at[pl.ds(s * NSL, NSL)])
    plsc.subcore_barrier()
    pltpu.sync_copy(denom_sh, denom_v)

    h0 = c * (EPT // 2)

    def p3(k, _):
        sl = pl.ds(h0 + k * 16, 16)
        dn = plsc.load_gather(denom_v, [dst_v[sl]])
        aexp_v[sl] = aexp_v[sl] / (dn + 1e-12)
        return 0
    lax.fori_loop(0, EPT // 2 // 16, p3, 0)
    pltpu.sync_copy(aexp_v.at[pl.ds(h0, EPT // 2)],
                    an_h.at[pl.ds(eb + h0, EPT // 2)])


_sc_scalar = pl.kernel(
    _sc_scalar_body,
    out_type=jax.ShapeDtypeStruct((E,), jnp.float32),
    mesh=_MESH,
    compiler_params=_SC_PARAMS,
    scratch_types=[
        pltpu.VMEM((N,), jnp.float32),
        pltpu.VMEM((N,), jnp.float32),
        pltpu.VMEM((NPAD,), jnp.float32),
        pltpu.VMEM((EPT,), jnp.int32),
        pltpu.VMEM((EPT,), jnp.int32),
        pltpu.VMEM((EPT,), jnp.float32),
        pltpu.VMEM((16,), jnp.float32),
        pltpu.VMEM((NS, 16), jnp.float32),
        pltpu.VMEM((NSL,), jnp.float32),
        pltpu.VMEM((NS, NSL), jnp.float32),
        pltpu.VMEM_SHARED((NS, 16), jnp.float32),
        pltpu.VMEM_SHARED((NS, NPAD), jnp.float32),
        pltpu.VMEM_SHARED((NPAD,), jnp.float32),
    ],
)


def _sc_msg_body(xs_h, src_h, dst_h, an_h, out_h,
                 xs_v, out_v, sb0, db0, wb0, sb1, db1, wb1, sem0, sem1):
    c = lax.axis_index("c")
    s = lax.axis_index("s")
    r0 = (s * NC + c) * CPT

    pltpu.sync_copy(xs_h.at[pl.ds(r0, CPT)], xs_v)

    def zero(k, _):
        sl = pl.ds(k * 16, 16)
        for j in range(CPT):
            out_v[j, sl] = jnp.zeros((16,), jnp.float32)
        return 0
    lax.fori_loop(0, N // 16, zero, 0)

    def fire(t, bufs, sem):
        off = t * CH
        pltpu.async_copy(src_h.at[pl.ds(off, CH)], bufs[0], sem)
        pltpu.async_copy(dst_h.at[pl.ds(off, CH)], bufs[1], sem)
        pltpu.async_copy(an_h.at[pl.ds(off, CH)], bufs[2], sem)

    def drain(t, bufs, sem):
        off = t * CH
        pltpu.make_async_copy(src_h.at[pl.ds(off, CH)], bufs[0], sem).wait()
        pltpu.make_async_copy(dst_h.at[pl.ds(off, CH)], bufs[1], sem).wait()
        pltpu.make_async_copy(an_h.at[pl.ds(off, CH)], bufs[2], sem).wait()

    def process(bufs):
        sbuf, dbuf, wbuf = bufs

        def inner(k, _):
            sl = pl.ds(k * 16, 16)
            s16 = sbuf[sl]
            d16 = dbuf[sl]
            w = wbuf[sl]
            for j in range(CPT):
                jv = jnp.full((16,), j, jnp.int32)
                v = plsc.load_gather(xs_v, [jv, s16])
                plsc.addupdate_scatter(out_v, [jv, d16], v * w)
            return 0
        lax.fori_loop(0, CH // 16, inner, 0)

    bufs0 = (sb0, db0, wb0)
    bufs1 = (sb1, db1, wb1)
    fire(0, bufs0, sem0)
    fire(1, bufs1, sem1)

    def step(t, _):
        @pl.when(t % 2 == 0)
        def _():
            drain(t, bufs0, sem0)
            process(bufs0)

            @pl.when(t + 2 < NT)
            def _():
                fire(t + 2, bufs0, sem0)

        @pl.when(t % 2 == 1)
        def _():
            drain(t, bufs1, sem1)
            process(bufs1)

            @pl.when(t + 2 < NT)
            def _():
                fire(t + 2, bufs1, sem1)
        return 0
    lax.fori_loop(0, NT, step, 0)

    pltpu.sync_copy(out_v, out_h.at[pl.ds(r0, CPT)])


_sc_msg = pl.kernel(
    _sc_msg_body,
    out_type=jax.ShapeDtypeStruct((D, N), jnp.float32),
    mesh=_MESH,
    compiler_params=_SC_PARAMS,
    scratch_types=[
        pltpu.VMEM((CPT, N), jnp.float32),
        pltpu.VMEM((CPT, N), jnp.float32),
        pltpu.VMEM((CH,), jnp.int32),
        pltpu.VMEM((CH,), jnp.int32),
        pltpu.VMEM((CH,), jnp.float32),
        pltpu.VMEM((CH,), jnp.int32),
        pltpu.VMEM((CH,), jnp.int32),
        pltpu.VMEM((CH,), jnp.float32),
        pltpu.SemaphoreType.DMA,
        pltpu.SemaphoreType.DMA,
    ],
)


def kernel(x, edge_index, edge_attr, W_src, W_dst, att, W_edge, bias, W_mid):
    src = edge_index[0].astype(jnp.int32)
    dst = edge_index[1].astype(jnp.int32)
    att_s = att[:, 0, :D]
    att_d = att[:, 0, D:]
    eye = jnp.eye(D, dtype=jnp.float32)

    e_pre = _tc_edge_pre(edge_attr, att_s, W_edge)
    H = _tc_transpose(x, eye)

    def conv_t(i, Hin):
        XS, ss, sd = _tc_matmul(Hin, W_src[i], W_dst[i],
                                att_s[i:i + 1], att_d[i:i + 1])
        an = _sc_scalar(src, dst, ss.reshape(N), sd.reshape(N), e_pre[i])
        return _sc_msg(XS, src, dst, an)

    O0 = conv_t(0, H)
    x0T = _tc_bias(O0, bias[0].reshape(D, 1))
    h = x0T
    for i in range(NMID):
        Oi = conv_t(i + 1, h)
        beta = math.log(THETA / (i + 1) + 1.0)
        h = _tc_resid(Oi, x0T, W_mid[i], bias[i + 1].reshape(D, 1), beta)
    O4 = conv_t(4, h)
    return _tc_final(O4, bias[4].reshape(1, D), eye)
